# Initial kernel scaffold; baseline (speedup 1.0000x reference)
import jax
import jax.numpy as jnp
from jax.experimental import pallas as pl


def kernel(x, edge_index, W1, b1, g1, be1, W2, b2, eps, go, bo):
    raise NotImplementedError("write your pallas kernel here")


# speedup vs baseline: 4.8770x; 4.8770x over previous
import functools

import jax
import jax.numpy as jnp
from jax import lax
from jax.experimental import pallas as pl
from jax.experimental.pallas import tpu as pltpu
from jax.experimental.pallas import tpu_sc as plsc

DEPTH = 3
N = 10000
D = 128
H = 256
E = 320000

_NUM_CORES = 2
_NUM_TILES = 16
_CHUNK = 80
_EDGES_PER_TILE = E // (_NUM_CORES * _NUM_TILES)
_N_CHUNKS = _EDGES_PER_TILE // _CHUNK
_ROWS_MAIN = 624
_ROWS_TAIL = N - _NUM_TILES * _ROWS_MAIN


def _sc_agg_body(h_hbm, src_hbm, dst_hbm, out_hbm, agg, src_v, dst_v, rows_v, sem):
    c = lax.axis_index("c")
    s = lax.axis_index("s")
    r0 = s * _ROWS_MAIN
    pltpu.sync_copy(h_hbm.at[pl.ds(r0, _ROWS_MAIN)],
                    agg.at[pl.ds(r0, _ROWS_MAIN)])

    @pl.when(s == _NUM_TILES - 1)
    def _init_tail():
        pltpu.sync_copy(h_hbm.at[pl.ds(_NUM_TILES * _ROWS_MAIN, _ROWS_TAIL)],
                        agg.at[pl.ds(_NUM_TILES * _ROWS_MAIN, _ROWS_TAIL)])

    plsc.subcore_barrier()

    base = c * (E // _NUM_CORES) + s * _EDGES_PER_TILE

    def body(j, carry):
        off = base + j * _CHUNK
        pltpu.sync_copy(src_hbm.at[pl.ds(off, _CHUNK)], src_v)
        pltpu.sync_copy(dst_hbm.at[pl.ds(off, _CHUNK)], dst_v)
        pltpu.async_copy(h_hbm.at[src_v], rows_v, sem).wait()
        pltpu.sync_copy(rows_v, agg.at[dst_v], add=True)
        return carry

    lax.fori_loop(0, _N_CHUNKS, body, 0)
    plsc.subcore_barrier()
    pltpu.sync_copy(agg.at[pl.ds(r0, _ROWS_MAIN)],
                    out_hbm.at[pl.ds(c * N + r0, _ROWS_MAIN)])

    @pl.when(s == _NUM_TILES - 1)
    def _out_tail():
        pltpu.sync_copy(agg.at[pl.ds(_NUM_TILES * _ROWS_MAIN, _ROWS_TAIL)],
                        out_hbm.at[pl.ds(c * N + _NUM_TILES * _ROWS_MAIN, _ROWS_TAIL)])


@functools.lru_cache(maxsize=1)
def _sc_aggregate():
    return pl.kernel(
        _sc_agg_body,
        out_type=jax.ShapeDtypeStruct((_NUM_CORES * N, D), jnp.float32),
        mesh=plsc.VectorSubcoreMesh(core_axis_name="c", subcore_axis_name="s"),
        scratch_types=[
            pltpu.VMEM_SHARED((N, D), jnp.float32),
            pltpu.VMEM((_CHUNK,), jnp.int32),
            pltpu.VMEM((_CHUNK,), jnp.int32),
            pltpu.VMEM((_CHUNK, D), jnp.float32),
            pltpu.SemaphoreType.DMA,
        ],
    )


def _mlp_body(h_ref, agg_ref, m_ref, eps_ref, w1_ref, b1_ref, g1_ref, be1_ref,
              w2_ref, b2_ref, go_ref, bo_ref, hout_ref, mout_ref):
    eps = eps_ref[0]
    z = (eps - 1.0) * h_ref[...] + agg_ref[0] + agg_ref[1]
    y = jnp.dot(z, w1_ref[...], preferred_element_type=jnp.float32) + b1_ref[...]
    mu = jnp.mean(y, axis=0, keepdims=True)
    yc = y - mu
    var = jnp.mean(yc * yc, axis=0, keepdims=True)
    y = g1_ref[...] * yc * lax.rsqrt(var + 1e-5) + be1_ref[...]
    y = jnp.maximum(y, 0.0)
    w = jnp.dot(y, w2_ref[...], preferred_element_type=jnp.float32) + b2_ref[...]
    mu2 = jnp.mean(w, axis=0, keepdims=True)
    wc = w - mu2
    var2 = jnp.mean(wc * wc, axis=0, keepdims=True)
    w = go_ref[...] * wc * lax.rsqrt(var2 + 1e-5) + bo_ref[...]
    hnew = jnp.maximum(w, 0.0)
    hout_ref[...] = hnew
    mout_ref[...] = jnp.maximum(m_ref[...], hnew)


def _mlp_call(h, agg2, m, eps_i, w1, b1, g1, be1, w2, b2, go, bo):
    return pl.pallas_call(
        _mlp_body,
        out_shape=(
            jax.ShapeDtypeStruct((N, D), jnp.float32),
            jax.ShapeDtypeStruct((N, D), jnp.float32),
        ),
        in_specs=[
            pl.BlockSpec(memory_space=pltpu.VMEM),
            pl.BlockSpec(memory_space=pltpu.VMEM),
            pl.BlockSpec(memory_space=pltpu.VMEM),
            pl.BlockSpec(memory_space=pltpu.SMEM),
        ] + [pl.BlockSpec(memory_space=pltpu.VMEM)] * 8,
    )(h, agg2, m, eps_i, w1, b1, g1, be1, w2, b2, go, bo)


def kernel(x, edge_index, W1, b1, g1, be1, W2, b2, eps, go, bo):
    src = edge_index[0]
    dst = edge_index[1]
    h = x
    m = jnp.zeros_like(x)
    for i in range(DEPTH):
        agg2 = _sc_aggregate()(h, src, dst).reshape(_NUM_CORES, N, D)
        h, m = _mlp_call(
            h, agg2, m,
            eps[i].reshape(1),
            W1[i], b1[i].reshape(1, H), g1[i].reshape(1, H), be1[i].reshape(1, H),
            W2[i], b2[i].reshape(1, D), go[i].reshape(1, D), bo[i].reshape(1, D),
        )
    return m


# speedup vs baseline: 10.6799x; 2.1898x over previous
import functools

import jax
import jax.numpy as jnp
from jax import lax
from jax.experimental import pallas as pl
from jax.experimental.pallas import tpu as pltpu
from jax.experimental.pallas import tpu_sc as plsc

DEPTH = 3
N = 10000
D = 128
H = 256
E = 320000

_NUM_CORES = 2
_NUM_TILES = 16
_NUM_WORKERS = _NUM_CORES * _NUM_TILES
_CHUNK = 64
_NCH = 160
_BLK = 16
_NPAIR = _NCH // (2 * _BLK)
_NSLOTS = 4
_AHEAD = 2
_PAD_E = _NUM_WORKERS * _NCH * _CHUNK
_NROWS_AGG = N + 16
_ROWS_MAIN = 624
_ROWS_TAIL = N - _NUM_TILES * _ROWS_MAIN


def _sc_agg_body(h_hbm, src_hbm, dst_hbm, out_hbm, agg,
                 src_a, src_b, dst_a, dst_b,
                 rows0, rows1, rows2, rows3,
                 ia, ib, g0, g1, g2, g3, s0, s1, s2, s3):
    c = lax.axis_index("c")
    s = lax.axis_index("s")
    w = c * _NUM_TILES + s
    rows = (rows0, rows1, rows2, rows3)
    gsem = (g0, g1, g2, g3)
    ssem = (s0, s1, s2, s3)

    def load_blk(kk, sb, db, sem):
        kb = w * _NCH + kk * _BLK
        pltpu.async_copy(src_hbm.at[pl.ds(kb, _BLK)], sb, sem)
        pltpu.async_copy(dst_hbm.at[pl.ds(kb, _BLK)], db, sem)

    def wait_blk(sb, db, sem):
        pltpu.make_async_copy(src_hbm.at[pl.ds(0, _BLK)], sb, sem).wait()
        pltpu.make_async_copy(dst_hbm.at[pl.ds(0, _BLK)], db, sem).wait()

    def gissue(sb, j, slot):
        pltpu.async_copy(h_hbm.at[sb.at[j]], rows[slot], gsem[slot])

    def gwait(sb, j, slot):
        pltpu.make_async_copy(h_hbm.at[sb.at[j]], rows[slot], gsem[slot]).wait()

    def sissue(db, j, slot):
        pltpu.async_copy(rows[slot], agg.at[db.at[j]], ssem[slot], add=True)

    def swait(db, j, slot):
        pltpu.make_async_copy(rows[slot], agg.at[db.at[j]], ssem[slot]).wait()

    load_blk(0, src_a, dst_a, ia)
    r0 = s * _ROWS_MAIN
    pltpu.sync_copy(h_hbm.at[pl.ds(r0, _ROWS_MAIN)],
                    agg.at[pl.ds(r0, _ROWS_MAIN)])

    @pl.when(s == _NUM_TILES - 1)
    def _init_tail():
        pltpu.sync_copy(h_hbm.at[pl.ds(_NUM_TILES * _ROWS_MAIN, _ROWS_TAIL)],
                        agg.at[pl.ds(_NUM_TILES * _ROWS_MAIN, _ROWS_TAIL)])

    wait_blk(src_a, dst_a, ia)
    for b in range(_AHEAD):
        gissue(src_a, b, b)
    plsc.subcore_barrier()

    def emit_block(k2, is_a):
        cur_s, cur_d = (src_a, dst_a) if is_a else (src_b, dst_b)
        nxt_s = src_b if is_a else src_a
        for j in range(_BLK):
            slot = j % _NSLOTS
            gwait(cur_s, j, slot)
            sissue(cur_d, j, slot)
            if j == 2:
                if is_a:
                    load_blk(2 * k2 + 1, src_b, dst_b, ib)
                else:
                    @pl.when(k2 < _NPAIR - 1)
                    def _load_next_a():
                        load_blk(2 * k2 + 2, src_a, dst_a, ia)
            aslot = (j + _AHEAD) % _NSLOTS
            if j < _BLK - _AHEAD:
                if is_a and j < _AHEAD:
                    @pl.when(k2 > 0)
                    def _sw():
                        swait(cur_d, j, aslot)
                else:
                    swait(cur_d, j, aslot)
                gissue(cur_s, j + _AHEAD, aslot)
            else:
                jn = j + _AHEAD - _BLK
                if is_a:
                    if j == _BLK - _AHEAD:
                        wait_blk(src_b, dst_b, ib)
                    swait(cur_d, j, aslot)
                    gissue(nxt_s, jn, aslot)
                else:
                    swait(cur_d, j, aslot)

                    @pl.when(k2 < _NPAIR - 1)
                    def _ahead_next_pair():
                        if j == _BLK - _AHEAD:
                            wait_blk(src_a, dst_a, ia)
                        gissue(nxt_s, jn, aslot)
        return None

    def pair_body(k2, carry):
        emit_block(k2, True)
        emit_block(k2, False)
        return carry

    lax.fori_loop(0, _NPAIR, pair_body, 0)

    for j in range(_BLK - _AHEAD, _BLK):
        swait(dst_b, j, j % _NSLOTS)
    plsc.subcore_barrier()
    pltpu.sync_copy(agg.at[pl.ds(r0, _ROWS_MAIN)],
                    out_hbm.at[pl.ds(c * N + r0, _ROWS_MAIN)])

    @pl.when(s == _NUM_TILES - 1)
    def _out_tail():
        pltpu.sync_copy(agg.at[pl.ds(_NUM_TILES * _ROWS_MAIN, _ROWS_TAIL)],
                        out_hbm.at[pl.ds(c * N + _NUM_TILES * _ROWS_MAIN,
                                         _ROWS_TAIL)])


@functools.lru_cache(maxsize=1)
def _sc_aggregate():
    return pl.kernel(
        _sc_agg_body,
        out_type=jax.ShapeDtypeStruct((_NUM_CORES * N, D), jnp.float32),
        mesh=plsc.VectorSubcoreMesh(core_axis_name="c", subcore_axis_name="s"),
        scratch_types=[
            pltpu.VMEM_SHARED((_NROWS_AGG, D), jnp.float32),
        ] + [pltpu.VMEM((_BLK, _CHUNK), jnp.int32)] * 4
          + [pltpu.VMEM((_CHUNK, D), jnp.float32)] * _NSLOTS
          + [pltpu.SemaphoreType.DMA] * (2 + 2 * _NSLOTS),
    )


def _mlp_body(h_ref, agg_ref, m_ref, eps_ref, w1_ref, b1_ref, g1_ref, be1_ref,
              w2_ref, b2_ref, go_ref, bo_ref, hout_ref, mout_ref):
    eps = eps_ref[0]
    z = (eps - 1.0) * h_ref[...] + agg_ref[0] + agg_ref[1]
    y = jnp.dot(z, w1_ref[...], preferred_element_type=jnp.float32) + b1_ref[...]
    mu = jnp.mean(y, axis=0, keepdims=True)
    yc = y - mu
    var = jnp.mean(yc * yc, axis=0, keepdims=True)
    y = g1_ref[...] * yc * lax.rsqrt(var + 1e-5) + be1_ref[...]
    y = jnp.maximum(y, 0.0)
    w = jnp.dot(y, w2_ref[...], preferred_element_type=jnp.float32) + b2_ref[...]
    mu2 = jnp.mean(w, axis=0, keepdims=True)
    wc = w - mu2
    var2 = jnp.mean(wc * wc, axis=0, keepdims=True)
    w = go_ref[...] * wc * lax.rsqrt(var2 + 1e-5) + bo_ref[...]
    hnew = jnp.maximum(w, 0.0)
    hout_ref[...] = hnew
    mout_ref[...] = jnp.maximum(m_ref[...], hnew)


def _mlp_call(h, agg2, m, eps_i, w1, b1, g1, be1, w2, b2, go, bo):
    return pl.pallas_call(
        _mlp_body,
        out_shape=(
            jax.ShapeDtypeStruct((N, D), jnp.float32),
            jax.ShapeDtypeStruct((N, D), jnp.float32),
        ),
        in_specs=[
            pl.BlockSpec(memory_space=pltpu.VMEM),
            pl.BlockSpec(memory_space=pltpu.VMEM),
            pl.BlockSpec(memory_space=pltpu.VMEM),
            pl.BlockSpec(memory_space=pltpu.SMEM),
        ] + [pl.BlockSpec(memory_space=pltpu.VMEM)] * 8,
    )(h, agg2, m, eps_i, w1, b1, g1, be1, w2, b2, go, bo)


def kernel(x, edge_index, W1, b1, g1, be1, W2, b2, eps, go, bo):
    src = edge_index[0]
    dst = edge_index[1]
    npad = _PAD_E - E
    pad_iota = jnp.arange(npad, dtype=jnp.int32)
    src2 = jnp.concatenate([src, (pad_iota * 97) % N]).reshape(-1, _CHUNK)
    dst2 = jnp.concatenate([dst, N + (pad_iota % 16)]).reshape(-1, _CHUNK)
    h = x
    m = jnp.zeros_like(x)
    for i in range(DEPTH):
        agg2 = _sc_aggregate()(h, src2, dst2).reshape(_NUM_CORES, N, D)
        h, m = _mlp_call(
            h, agg2, m,
            eps[i].reshape(1),
            W1[i], b1[i].reshape(1, H), g1[i].reshape(1, H), be1[i].reshape(1, H),
            W2[i], b2[i].reshape(1, D), go[i].reshape(1, D), bo[i].reshape(1, D),
        )
    return m


# speedup vs baseline: 11.1164x; 1.0409x over previous
import functools

import jax
import jax.numpy as jnp
from jax import lax
from jax.experimental import pallas as pl
from jax.experimental.pallas import tpu as pltpu
from jax.experimental.pallas import tpu_sc as plsc

DEPTH = 3
N = 10000
D = 128
H = 256
E = 320000

_NUM_CORES = 2
_NUM_TILES = 16
_NUM_WORKERS = _NUM_CORES * _NUM_TILES
_CHUNK = 80
_NCH = 128
_BLK = 16
_NPAIR = _NCH // (2 * _BLK)
_NSLOTS = 4
_AHEAD = 2
_PAD_E = _NUM_WORKERS * _NCH * _CHUNK
_NROWS_AGG = N + 16
_ROWS_MAIN = 624
_ROWS_TAIL = N - _NUM_TILES * _ROWS_MAIN


def _sc_agg_body(h_hbm, src_hbm, dst_hbm, out_hbm, agg,
                 src_a, src_b, dst_a, dst_b,
                 rows0, rows1, rows2, rows3,
                 ia, ib, g0, g1, g2, g3, s0, s1, s2, s3):
    c = lax.axis_index("c")
    s = lax.axis_index("s")
    w = c * _NUM_TILES + s
    rows = (rows0, rows1, rows2, rows3)
    gsem = (g0, g1, g2, g3)
    ssem = (s0, s1, s2, s3)

    def load_blk(kk, sb, db, sem):
        kb = w * _NCH + kk * _BLK
        pltpu.async_copy(src_hbm.at[pl.ds(kb, _BLK)], sb, sem)
        pltpu.async_copy(dst_hbm.at[pl.ds(kb, _BLK)], db, sem)

    def wait_blk(sb, db, sem):
        pltpu.make_async_copy(src_hbm.at[pl.ds(0, _BLK)], sb, sem).wait()
        pltpu.make_async_copy(dst_hbm.at[pl.ds(0, _BLK)], db, sem).wait()

    def gissue(sb, j, slot):
        pltpu.async_copy(h_hbm.at[sb.at[j]], rows[slot], gsem[slot])

    def gwait(sb, j, slot):
        pltpu.make_async_copy(h_hbm.at[sb.at[j]], rows[slot], gsem[slot]).wait()

    def sissue(db, j, slot):
        pltpu.async_copy(rows[slot], agg.at[db.at[j]], ssem[slot], add=True)

    def swait(db, j, slot):
        pltpu.make_async_copy(rows[slot], agg.at[db.at[j]], ssem[slot]).wait()

    load_blk(0, src_a, dst_a, ia)
    r0 = s * _ROWS_MAIN
    pltpu.sync_copy(h_hbm.at[pl.ds(r0, _ROWS_MAIN)],
                    agg.at[pl.ds(r0, _ROWS_MAIN)])

    @pl.when(s == _NUM_TILES - 1)
    def _init_tail():
        pltpu.sync_copy(h_hbm.at[pl.ds(_NUM_TILES * _ROWS_MAIN, _ROWS_TAIL)],
                        agg.at[pl.ds(_NUM_TILES * _ROWS_MAIN, _ROWS_TAIL)])

    wait_blk(src_a, dst_a, ia)
    for b in range(_AHEAD):
        gissue(src_a, b, b)
    plsc.subcore_barrier()

    def emit_block(k2, is_a):
        cur_s, cur_d = (src_a, dst_a) if is_a else (src_b, dst_b)
        nxt_s = src_b if is_a else src_a
        for j in range(_BLK):
            slot = j % _NSLOTS
            gwait(cur_s, j, slot)
            sissue(cur_d, j, slot)
            if j == 2:
                if is_a:
                    load_blk(2 * k2 + 1, src_b, dst_b, ib)
                else:
                    @pl.when(k2 < _NPAIR - 1)
                    def _load_next_a():
                        load_blk(2 * k2 + 2, src_a, dst_a, ia)
            aslot = (j + _AHEAD) % _NSLOTS
            if j < _BLK - _AHEAD:
                if is_a and j < _AHEAD:
                    @pl.when(k2 > 0)
                    def _sw():
                        swait(cur_d, j, aslot)
                else:
                    swait(cur_d, j, aslot)
                gissue(cur_s, j + _AHEAD, aslot)
            else:
                jn = j + _AHEAD - _BLK
                if is_a:
                    if j == _BLK - _AHEAD:
                        wait_blk(src_b, dst_b, ib)
                    swait(cur_d, j, aslot)
                    gissue(nxt_s, jn, aslot)
                else:
                    swait(cur_d, j, aslot)

                    @pl.when(k2 < _NPAIR - 1)
                    def _ahead_next_pair():
                        if j == _BLK - _AHEAD:
                            wait_blk(src_a, dst_a, ia)
                        gissue(nxt_s, jn, aslot)
        return None

    def pair_body(k2, carry):
        emit_block(k2, True)
        emit_block(k2, False)
        return carry

    lax.fori_loop(0, _NPAIR, pair_body, 0)

    for j in range(_BLK - _AHEAD, _BLK):
        swait(dst_b, j, j % _NSLOTS)
    plsc.subcore_barrier()
    pltpu.sync_copy(agg.at[pl.ds(r0, _ROWS_MAIN)],
                    out_hbm.at[pl.ds(c * N + r0, _ROWS_MAIN)])

    @pl.when(s == _NUM_TILES - 1)
    def _out_tail():
        pltpu.sync_copy(agg.at[pl.ds(_NUM_TILES * _ROWS_MAIN, _ROWS_TAIL)],
                        out_hbm.at[pl.ds(c * N + _NUM_TILES * _ROWS_MAIN,
                                         _ROWS_TAIL)])


@functools.lru_cache(maxsize=1)
def _sc_aggregate():
    return pl.kernel(
        _sc_agg_body,
        out_type=jax.ShapeDtypeStruct((_NUM_CORES * N, D), jnp.float32),
        mesh=plsc.VectorSubcoreMesh(core_axis_name="c", subcore_axis_name="s"),
        scratch_types=[
            pltpu.VMEM_SHARED((_NROWS_AGG, D), jnp.float32),
        ] + [pltpu.VMEM((_BLK, _CHUNK), jnp.int32)] * 4
          + [pltpu.VMEM((_CHUNK, D), jnp.float32)] * _NSLOTS
          + [pltpu.SemaphoreType.DMA] * (2 + 2 * _NSLOTS),
    )


def _mlp_body(h_ref, agg_ref, m_ref, eps_ref, w1_ref, b1_ref, g1_ref, be1_ref,
              w2_ref, b2_ref, go_ref, bo_ref, hout_ref, mout_ref):
    eps = eps_ref[0]
    z = (eps - 1.0) * h_ref[...] + agg_ref[0] + agg_ref[1]
    y = jnp.dot(z, w1_ref[...], preferred_element_type=jnp.float32) + b1_ref[...]
    mu = jnp.mean(y, axis=0, keepdims=True)
    yc = y - mu
    var = jnp.mean(yc * yc, axis=0, keepdims=True)
    y = g1_ref[...] * yc * lax.rsqrt(var + 1e-5) + be1_ref[...]
    y = jnp.maximum(y, 0.0)
    w = jnp.dot(y, w2_ref[...], preferred_element_type=jnp.float32) + b2_ref[...]
    mu2 = jnp.mean(w, axis=0, keepdims=True)
    wc = w - mu2
    var2 = jnp.mean(wc * wc, axis=0, keepdims=True)
    w = go_ref[...] * wc * lax.rsqrt(var2 + 1e-5) + bo_ref[...]
    hnew = jnp.maximum(w, 0.0)
    hout_ref[...] = hnew
    mout_ref[...] = jnp.maximum(m_ref[...], hnew)


def _mlp_call(h, agg2, m, eps_i, w1, b1, g1, be1, w2, b2, go, bo):
    return pl.pallas_call(
        _mlp_body,
        out_shape=(
            jax.ShapeDtypeStruct((N, D), jnp.float32),
            jax.ShapeDtypeStruct((N, D), jnp.float32),
        ),
        in_specs=[
            pl.BlockSpec(memory_space=pltpu.VMEM),
            pl.BlockSpec(memory_space=pltpu.VMEM),
            pl.BlockSpec(memory_space=pltpu.VMEM),
            pl.BlockSpec(memory_space=pltpu.SMEM),
        ] + [pl.BlockSpec(memory_space=pltpu.VMEM)] * 8,
    )(h, agg2, m, eps_i, w1, b1, g1, be1, w2, b2, go, bo)


def kernel(x, edge_index, W1, b1, g1, be1, W2, b2, eps, go, bo):
    src = edge_index[0]
    dst = edge_index[1]
    npad = _PAD_E - E
    pad_iota = jnp.arange(npad, dtype=jnp.int32)
    src2 = jnp.concatenate([src, (pad_iota * 97) % N]).reshape(-1, _CHUNK)
    dst2 = jnp.concatenate([dst, N + (pad_iota % 16)]).reshape(-1, _CHUNK)
    h = x
    m = jnp.zeros_like(x)
    for i in range(DEPTH):
        agg2 = _sc_aggregate()(h, src2, dst2).reshape(_NUM_CORES, N, D)
        h, m = _mlp_call(
            h, agg2, m,
            eps[i].reshape(1),
            W1[i], b1[i].reshape(1, H), g1[i].reshape(1, H), be1[i].reshape(1, H),
            W2[i], b2[i].reshape(1, D), go[i].reshape(1, D), bo[i].reshape(1, D),
        )
    return m


# speedup vs baseline: 12.1172x; 1.0900x over previous
import functools

import jax
import jax.numpy as jnp
from jax import lax
from jax.experimental import pallas as pl
from jax.experimental.pallas import tpu as pltpu
from jax.experimental.pallas import tpu_sc as plsc

DEPTH = 3
N = 10000
D = 128
H = 256
E = 320000

_NUM_CORES = 2
_NUM_TILES = 16
_NUM_WORKERS = _NUM_CORES * _NUM_TILES
_CHUNK = 80
_NCH = 128
_BLK = 16
_NPAIR = _NCH // (2 * _BLK)
_NSLOTS = 4
_AHEAD = 3
_PAD_E = _NUM_WORKERS * _NCH * _CHUNK
_NROWS_AGG = N + 16
_ROWS_MAIN = 624
_ROWS_TAIL = N - _NUM_TILES * _ROWS_MAIN


def _sc_agg_body(h_hbm, src_hbm, dst_hbm, out_hbm, agg,
                 src_a, src_b, dst_a, dst_b,
                 rows0, rows1, rows2, rows3,
                 ia, ib, g0, g1, g2, g3, s0, s1, s2, s3):
    c = lax.axis_index("c")
    s = lax.axis_index("s")
    w = c * _NUM_TILES + s
    rows = (rows0, rows1, rows2, rows3)
    gsem = (g0, g1, g2, g3)
    ssem = (s0, s1, s2, s3)

    def load_blk(kk, sb, db, sem):
        kb = w * _NCH + kk * _BLK
        pltpu.async_copy(src_hbm.at[pl.ds(kb, _BLK)], sb, sem)
        pltpu.async_copy(dst_hbm.at[pl.ds(kb, _BLK)], db, sem)

    def wait_blk(sb, db, sem):
        pltpu.make_async_copy(src_hbm.at[pl.ds(0, _BLK)], sb, sem).wait()
        pltpu.make_async_copy(dst_hbm.at[pl.ds(0, _BLK)], db, sem).wait()

    def gissue(sb, j, slot):
        pltpu.async_copy(h_hbm.at[sb.at[j]], rows[slot], gsem[slot])

    def gwait(sb, j, slot):
        pltpu.make_async_copy(h_hbm.at[sb.at[j]], rows[slot], gsem[slot]).wait()

    def sissue(db, j, slot):
        pltpu.async_copy(rows[slot], agg.at[db.at[j]], ssem[slot], add=True)

    def swait(db, j, slot):
        pltpu.make_async_copy(rows[slot], agg.at[db.at[j]], ssem[slot]).wait()

    load_blk(0, src_a, dst_a, ia)
    r0 = s * _ROWS_MAIN
    pltpu.sync_copy(h_hbm.at[pl.ds(r0, _ROWS_MAIN)],
                    agg.at[pl.ds(r0, _ROWS_MAIN)])

    @pl.when(s == _NUM_TILES - 1)
    def _init_tail():
        pltpu.sync_copy(h_hbm.at[pl.ds(_NUM_TILES * _ROWS_MAIN, _ROWS_TAIL)],
                        agg.at[pl.ds(_NUM_TILES * _ROWS_MAIN, _ROWS_TAIL)])

    wait_blk(src_a, dst_a, ia)
    for b in range(_AHEAD):
        gissue(src_a, b, b)
    plsc.subcore_barrier()

    def emit_block(k2, is_a):
        cur_s, cur_d = (src_a, dst_a) if is_a else (src_b, dst_b)
        nxt_s = src_b if is_a else src_a
        for j in range(_BLK):
            slot = j % _NSLOTS
            gwait(cur_s, j, slot)
            sissue(cur_d, j, slot)
            if j == 2:
                if is_a:
                    load_blk(2 * k2 + 1, src_b, dst_b, ib)
                else:
                    @pl.when(k2 < _NPAIR - 1)
                    def _load_next_a():
                        load_blk(2 * k2 + 2, src_a, dst_a, ia)
            aslot = (j + _AHEAD) % _NSLOTS
            if j < _BLK - _AHEAD:
                if is_a and j < _NSLOTS - _AHEAD:
                    @pl.when(k2 > 0)
                    def _sw():
                        swait(cur_d, j, aslot)
                else:
                    swait(cur_d, j, aslot)
                gissue(cur_s, j + _AHEAD, aslot)
            else:
                jn = j + _AHEAD - _BLK
                if is_a:
                    if j == _BLK - _AHEAD:
                        wait_blk(src_b, dst_b, ib)
                    swait(cur_d, j, aslot)
                    gissue(nxt_s, jn, aslot)
                else:
                    swait(cur_d, j, aslot)

                    @pl.when(k2 < _NPAIR - 1)
                    def _ahead_next_pair():
                        if j == _BLK - _AHEAD:
                            wait_blk(src_a, dst_a, ia)
                        gissue(nxt_s, jn, aslot)
        return None

    def pair_body(k2, carry):
        emit_block(k2, True)
        emit_block(k2, False)
        return carry

    lax.fori_loop(0, _NPAIR, pair_body, 0)

    for j in range(_BLK - (_NSLOTS - _AHEAD), _BLK):
        swait(dst_b, j, j % _NSLOTS)
    plsc.subcore_barrier()
    pltpu.sync_copy(agg.at[pl.ds(r0, _ROWS_MAIN)],
                    out_hbm.at[pl.ds(c * N + r0, _ROWS_MAIN)])

    @pl.when(s == _NUM_TILES - 1)
    def _out_tail():
        pltpu.sync_copy(agg.at[pl.ds(_NUM_TILES * _ROWS_MAIN, _ROWS_TAIL)],
                        out_hbm.at[pl.ds(c * N + _NUM_TILES * _ROWS_MAIN,
                                         _ROWS_TAIL)])


@functools.lru_cache(maxsize=1)
def _sc_aggregate():
    return pl.kernel(
        _sc_agg_body,
        out_type=jax.ShapeDtypeStruct((_NUM_CORES * N, D), jnp.float32),
        mesh=plsc.VectorSubcoreMesh(core_axis_name="c", subcore_axis_name="s"),
        scratch_types=[
            pltpu.VMEM_SHARED((_NROWS_AGG, D), jnp.float32),
        ] + [pltpu.VMEM((_BLK, _CHUNK), jnp.int32)] * 4
          + [pltpu.VMEM((_CHUNK, D), jnp.float32)] * _NSLOTS
          + [pltpu.SemaphoreType.DMA] * (2 + 2 * _NSLOTS),
    )


def _mlp_body(h_ref, agg_ref, m_ref, eps_ref, w1_ref, b1_ref, g1_ref, be1_ref,
              w2_ref, b2_ref, go_ref, bo_ref, hout_ref, mout_ref):
    eps = eps_ref[0]
    z = (eps - 1.0) * h_ref[...] + agg_ref[0] + agg_ref[1]
    y = jnp.dot(z, w1_ref[...], preferred_element_type=jnp.float32) + b1_ref[...]
    mu = jnp.mean(y, axis=0, keepdims=True)
    yc = y - mu
    var = jnp.mean(yc * yc, axis=0, keepdims=True)
    y = g1_ref[...] * yc * lax.rsqrt(var + 1e-5) + be1_ref[...]
    y = jnp.maximum(y, 0.0)
    w = jnp.dot(y, w2_ref[...], preferred_element_type=jnp.float32) + b2_ref[...]
    mu2 = jnp.mean(w, axis=0, keepdims=True)
    wc = w - mu2
    var2 = jnp.mean(wc * wc, axis=0, keepdims=True)
    w = go_ref[...] * wc * lax.rsqrt(var2 + 1e-5) + bo_ref[...]
    hnew = jnp.maximum(w, 0.0)
    hout_ref[...] = hnew
    mout_ref[...] = jnp.maximum(m_ref[...], hnew)


def _mlp_call(h, agg2, m, eps_i, w1, b1, g1, be1, w2, b2, go, bo):
    return pl.pallas_call(
        _mlp_body,
        out_shape=(
            jax.ShapeDtypeStruct((N, D), jnp.float32),
            jax.ShapeDtypeStruct((N, D), jnp.float32),
        ),
        in_specs=[
            pl.BlockSpec(memory_space=pltpu.VMEM),
            pl.BlockSpec(memory_space=pltpu.VMEM),
            pl.BlockSpec(memory_space=pltpu.VMEM),
            pl.BlockSpec(memory_space=pltpu.SMEM),
        ] + [pl.BlockSpec(memory_space=pltpu.VMEM)] * 8,
    )(h, agg2, m, eps_i, w1, b1, g1, be1, w2, b2, go, bo)


def kernel(x, edge_index, W1, b1, g1, be1, W2, b2, eps, go, bo):
    src = edge_index[0]
    dst = edge_index[1]
    npad = _PAD_E - E
    pad_iota = jnp.arange(npad, dtype=jnp.int32)
    src2 = jnp.concatenate([src, (pad_iota * 97) % N]).reshape(-1, _CHUNK)
    dst2 = jnp.concatenate([dst, N + (pad_iota % 16)]).reshape(-1, _CHUNK)
    h = x
    m = jnp.zeros_like(x)
    for i in range(DEPTH):
        agg2 = _sc_aggregate()(h, src2, dst2).reshape(_NUM_CORES, N, D)
        h, m = _mlp_call(
            h, agg2, m,
            eps[i].reshape(1),
            W1[i], b1[i].reshape(1, H), g1[i].reshape(1, H), be1[i].reshape(1, H),
            W2[i], b2[i].reshape(1, D), go[i].reshape(1, D), bo[i].reshape(1, D),
        )
    return m


# speedup vs baseline: 12.6836x; 1.0467x over previous
import functools

import jax
import jax.numpy as jnp
from jax import lax
from jax.experimental import pallas as pl
from jax.experimental.pallas import tpu as pltpu
from jax.experimental.pallas import tpu_sc as plsc

DEPTH = 3
N = 10000
D = 128
H = 256
E = 320000

_NUM_CORES = 2
_NUM_TILES = 16
_NUM_WORKERS = _NUM_CORES * _NUM_TILES
_CHUNK = 80
_NCH = 128
_BLK = 16
_NPAIR = _NCH // (2 * _BLK)
_NSLOTS = 4
_AHEAD = 3
_PAD_E = _NUM_WORKERS * _NCH * _CHUNK
_NROWS_AGG = N + 16
_ROWS_MAIN = 624
_ROWS_TAIL = N - _NUM_TILES * _ROWS_MAIN


def _sc_agg_body(h_hbm, src_hbm, dst_hbm, out_hbm, agg,
                 src_a, src_b, dst_a, dst_b,
                 rows0, rows1, rows2, rows3,
                 ia, ib, g0, g1, g2, g3, s0, s1, s2, s3):
    c = lax.axis_index("c")
    s = lax.axis_index("s")
    w = c * _NUM_TILES + s
    rows = (rows0, rows1, rows2, rows3)
    gsem = (g0, g1, g2, g3)
    ssem = (s0, s1, s2, s3)

    def load_blk(kk, sb, db, sem):
        kb = w * _NCH + kk * _BLK
        pltpu.async_copy(src_hbm.at[pl.ds(kb, _BLK)], sb, sem)
        pltpu.async_copy(dst_hbm.at[pl.ds(kb, _BLK)], db, sem)

    def wait_blk(sb, db, sem):
        pltpu.make_async_copy(src_hbm.at[pl.ds(0, _BLK)], sb, sem).wait()
        pltpu.make_async_copy(dst_hbm.at[pl.ds(0, _BLK)], db, sem).wait()

    def gissue(sb, j, slot):
        pltpu.async_copy(h_hbm.at[sb.at[j]], rows[slot], gsem[slot])

    def gwait(sb, j, slot):
        pltpu.make_async_copy(h_hbm.at[sb.at[j]], rows[slot], gsem[slot]).wait()

    def sissue(db, j, slot):
        pltpu.async_copy(rows[slot], agg.at[db.at[j]], ssem[slot], add=True)

    def swait(db, j, slot):
        pltpu.make_async_copy(rows[slot], agg.at[db.at[j]], ssem[slot]).wait()

    load_blk(0, src_a, dst_a, ia)
    r0 = s * _ROWS_MAIN
    pltpu.sync_copy(h_hbm.at[pl.ds(r0, _ROWS_MAIN)],
                    agg.at[pl.ds(r0, _ROWS_MAIN)])

    @pl.when(s == _NUM_TILES - 1)
    def _init_tail():
        pltpu.sync_copy(h_hbm.at[pl.ds(_NUM_TILES * _ROWS_MAIN, _ROWS_TAIL)],
                        agg.at[pl.ds(_NUM_TILES * _ROWS_MAIN, _ROWS_TAIL)])

    wait_blk(src_a, dst_a, ia)
    for b in range(_AHEAD):
        gissue(src_a, b, b)
    plsc.subcore_barrier()

    def emit_block(k2, is_a):
        cur_s, cur_d = (src_a, dst_a) if is_a else (src_b, dst_b)
        nxt_s = src_b if is_a else src_a
        for j in range(_BLK):
            slot = j % _NSLOTS
            gwait(cur_s, j, slot)
            sissue(cur_d, j, slot)
            if j == 2:
                if is_a:
                    load_blk(2 * k2 + 1, src_b, dst_b, ib)
                else:
                    @pl.when(k2 < _NPAIR - 1)
                    def _load_next_a():
                        load_blk(2 * k2 + 2, src_a, dst_a, ia)
            aslot = (j + _AHEAD) % _NSLOTS
            if j < _BLK - _AHEAD:
                if is_a and j < _NSLOTS - _AHEAD:
                    @pl.when(k2 > 0)
                    def _sw():
                        swait(cur_d, j, aslot)
                else:
                    swait(cur_d, j, aslot)
                gissue(cur_s, j + _AHEAD, aslot)
            else:
                jn = j + _AHEAD - _BLK
                if is_a:
                    if j == _BLK - _AHEAD:
                        wait_blk(src_b, dst_b, ib)
                    swait(cur_d, j, aslot)
                    gissue(nxt_s, jn, aslot)
                else:
                    swait(cur_d, j, aslot)

                    @pl.when(k2 < _NPAIR - 1)
                    def _ahead_next_pair():
                        if j == _BLK - _AHEAD:
                            wait_blk(src_a, dst_a, ia)
                        gissue(nxt_s, jn, aslot)
        return None

    def pair_body(k2, carry):
        emit_block(k2, True)
        emit_block(k2, False)
        return carry

    lax.fori_loop(0, _NPAIR, pair_body, 0)

    for j in range(_BLK - (_NSLOTS - _AHEAD), _BLK):
        swait(dst_b, j, j % _NSLOTS)
    plsc.subcore_barrier()
    pltpu.sync_copy(agg.at[pl.ds(r0, _ROWS_MAIN)],
                    out_hbm.at[pl.ds(c * N + r0, _ROWS_MAIN)])

    @pl.when(s == _NUM_TILES - 1)
    def _out_tail():
        pltpu.sync_copy(agg.at[pl.ds(_NUM_TILES * _ROWS_MAIN, _ROWS_TAIL)],
                        out_hbm.at[pl.ds(c * N + _NUM_TILES * _ROWS_MAIN,
                                         _ROWS_TAIL)])


@functools.lru_cache(maxsize=1)
def _sc_aggregate():
    return pl.kernel(
        _sc_agg_body,
        out_type=jax.ShapeDtypeStruct((_NUM_CORES * N, D), jnp.float32),
        mesh=plsc.VectorSubcoreMesh(core_axis_name="c", subcore_axis_name="s"),
        scratch_types=[
            pltpu.VMEM_SHARED((_NROWS_AGG, D), jnp.float32),
        ] + [pltpu.VMEM((_BLK, _CHUNK), jnp.int32)] * 4
          + [pltpu.VMEM((_CHUNK, D), jnp.float32)] * _NSLOTS
          + [pltpu.SemaphoreType.DMA] * (2 + 2 * _NSLOTS),
    )


def _bn(y, gamma, beta):
    mu = jnp.mean(y, axis=0, keepdims=True)
    ms = jnp.mean(y * y, axis=0, keepdims=True)
    var = ms - mu * mu
    return gamma * (y - mu) * lax.rsqrt(var + 1e-5) + beta


def _mlp_core(h, agg0, agg1, eps, w1, b1, g1, be1, w2, b2, go, bo):
    z = (eps - 1.0) * h + agg0 + agg1
    y = jnp.dot(z, w1, preferred_element_type=jnp.float32) + b1
    y = jnp.maximum(_bn(y, g1, be1), 0.0)
    w = jnp.dot(y, w2, preferred_element_type=jnp.float32) + b2
    return jnp.maximum(_bn(w, go, bo), 0.0)


def _mlp_body_first(h_ref, agg_ref, eps_ref, w1_ref, b1_ref, g1_ref, be1_ref,
                    w2_ref, b2_ref, go_ref, bo_ref, hout_ref):
    hout_ref[...] = _mlp_core(
        h_ref[...], agg_ref[0], agg_ref[1], eps_ref[0], w1_ref[...],
        b1_ref[...], g1_ref[...], be1_ref[...], w2_ref[...], b2_ref[...],
        go_ref[...], bo_ref[...])


def _mlp_body_mid(h_ref, agg_ref, eps_ref, w1_ref, b1_ref, g1_ref, be1_ref,
                  w2_ref, b2_ref, go_ref, bo_ref, hout_ref, mout_ref):
    h = h_ref[...]
    hnew = _mlp_core(
        h, agg_ref[0], agg_ref[1], eps_ref[0], w1_ref[...], b1_ref[...],
        g1_ref[...], be1_ref[...], w2_ref[...], b2_ref[...], go_ref[...],
        bo_ref[...])
    hout_ref[...] = hnew
    mout_ref[...] = jnp.maximum(h, hnew)


def _mlp_body_last(h_ref, agg_ref, m_ref, eps_ref, w1_ref, b1_ref, g1_ref,
                   be1_ref, w2_ref, b2_ref, go_ref, bo_ref, mout_ref):
    hnew = _mlp_core(
        h_ref[...], agg_ref[0], agg_ref[1], eps_ref[0], w1_ref[...],
        b1_ref[...], g1_ref[...], be1_ref[...], w2_ref[...], b2_ref[...],
        go_ref[...], bo_ref[...])
    mout_ref[...] = jnp.maximum(m_ref[...], hnew)


def _mlp_call(body, n_big_in, n_out, args):
    out = pl.pallas_call(
        body,
        out_shape=tuple(
            jax.ShapeDtypeStruct((N, D), jnp.float32) for _ in range(n_out)),
        in_specs=[pl.BlockSpec(memory_space=pltpu.VMEM)] * n_big_in +
                 [pl.BlockSpec(memory_space=pltpu.SMEM)] +
                 [pl.BlockSpec(memory_space=pltpu.VMEM)] * 8,
    )(*args)
    return out


def kernel(x, edge_index, W1, b1, g1, be1, W2, b2, eps, go, bo):
    src = edge_index[0]
    dst = edge_index[1]
    npad = _PAD_E - E
    pad_iota = jnp.arange(npad, dtype=jnp.int32)
    src2 = jnp.concatenate([src, (pad_iota * 97) % N]).reshape(-1, _CHUNK)
    dst2 = jnp.concatenate([dst, N + (pad_iota % 16)]).reshape(-1, _CHUNK)

    def layer_weights(i):
        return (eps[i].reshape(1), W1[i], b1[i].reshape(1, H),
                g1[i].reshape(1, H), be1[i].reshape(1, H), W2[i],
                b2[i].reshape(1, D), go[i].reshape(1, D), bo[i].reshape(1, D))

    agg = _sc_aggregate()(x, src2, dst2).reshape(_NUM_CORES, N, D)
    (h1,) = _mlp_call(_mlp_body_first, 2, 1, (x, agg) + layer_weights(0))
    agg = _sc_aggregate()(h1, src2, dst2).reshape(_NUM_CORES, N, D)
    h2, m2 = _mlp_call(_mlp_body_mid, 2, 2, (h1, agg) + layer_weights(1))
    agg = _sc_aggregate()(h2, src2, dst2).reshape(_NUM_CORES, N, D)
    (m,) = _mlp_call(_mlp_body_last, 3, 1, (h2, agg, m2) + layer_weights(2))
    return m
